# baseline (speedup 1.0000x reference)
import functools

import jax
import jax.numpy as jnp
from jax import lax
from jax.experimental import pallas as pl
from jax.experimental.pallas import tpu as pltpu
from jax.experimental.pallas import tpu_sc as plsc

N = 100000
C = 100
D = 128
BLK = 5000

NC = 2
NS = 16
NW = NC * NS
CHUNK = 128
FULL_CHUNKS = N // CHUNK
TAIL = N - FULL_CHUNKS * CHUNK
BASE_T = FULL_CHUNKS // NW
EXTRA_W = FULL_CHUNKS % NW


def _argmax_body(logits_ref, labels_ref, counts_ref):
    i = pl.program_id(0)
    x = logits_ref[...]
    lbl = jnp.argmax(x, axis=1).astype(jnp.int32)
    labels_ref[...] = lbl[:, None]
    iota = lax.broadcasted_iota(jnp.int32, (1, C), 1)
    onehot = (lbl[:, None] == iota).astype(jnp.float32)
    pcounts = jnp.sum(onehot, axis=0)[None, :]

    @pl.when(i == 0)
    def _():
        counts_ref[...] = pcounts

    @pl.when(i > 0)
    def _():
        counts_ref[...] += pcounts


def _sc_seg_body(feats_hbm, labels_hbm, zeros_hbm, out_hbm,
                 rows_v, idx_v, trows_v, tidx_v, acc_sh):
    cid = lax.axis_index("c")
    sid = lax.axis_index("s")
    w = sid * NC + cid

    @pl.when(sid == 0)
    def _():
        pltpu.sync_copy(zeros_hbm, acc_sh)

    plsc.subcore_barrier()

    def chunk_body(t, carry):
        base = (w + NW * t) * CHUNK
        pltpu.sync_copy(feats_hbm.at[pl.ds(base, CHUNK), :], rows_v)
        pltpu.sync_copy(labels_hbm.at[pl.ds(base, CHUNK)], idx_v)
        pltpu.sync_copy(rows_v, acc_sh.at[idx_v], add=True)
        return carry

    nt = jnp.where(w < EXTRA_W, BASE_T + 1, BASE_T)
    lax.fori_loop(0, nt, chunk_body, 0)

    @pl.when(w == NW - 1)
    def _():
        base = FULL_CHUNKS * CHUNK
        pltpu.sync_copy(feats_hbm.at[pl.ds(base, TAIL), :], trows_v)
        pltpu.sync_copy(labels_hbm.at[pl.ds(base, TAIL)], tidx_v)
        pltpu.sync_copy(trows_v, acc_sh.at[tidx_v], add=True)

    plsc.subcore_barrier()

    @pl.when(sid == 0)
    def _():
        pltpu.sync_copy(acc_sh, out_hbm.at[cid])


_sc_segment_sum = functools.partial(
    pl.kernel,
    _sc_seg_body,
    out_type=jax.ShapeDtypeStruct((NC, C, D), jnp.float32),
    mesh=plsc.VectorSubcoreMesh(core_axis_name="c", subcore_axis_name="s"),
    scratch_types=[
        pltpu.VMEM((CHUNK, D), jnp.float32),
        pltpu.VMEM((CHUNK,), jnp.int32),
        pltpu.VMEM((TAIL, D), jnp.float32),
        pltpu.VMEM((TAIL,), jnp.int32),
        pltpu.VMEM_SHARED((C, D), jnp.float32),
    ],
)()


def _loss_body(sums_ref, counts_ref, out_ref):
    cnt = counts_ref[0, :]
    sums = sums_ref[0] + sums_ref[1]
    recip = 1.0 / jnp.maximum(cnt, 1.0)
    means = sums * recip[:, None]
    sq = jnp.sum(means * means, axis=1, keepdims=True)
    norm = jnp.maximum(jnp.sqrt(sq), 1e-12)
    normed = means / norm
    cos = lax.dot_general(
        normed, normed, (((1,), (1,)), ((), ())),
        preferred_element_type=jnp.float32)
    present = (cnt > 0.0).astype(jnp.float32)
    pm = present[:, None] * present[None, :]
    ri = lax.broadcasted_iota(jnp.int32, (C, C), 0)
    ci = lax.broadcasted_iota(jnp.int32, (C, C), 1)
    offdiag = (ri != ci).astype(jnp.float32)
    loss = (1.0 - cos) * pm * offdiag
    out_ref[...] = jnp.sum(loss).reshape(1, 1)


def kernel(logits, img_feats):
    labels2d, counts = pl.pallas_call(
        _argmax_body,
        grid=(N // BLK,),
        in_specs=[pl.BlockSpec((BLK, C), lambda i: (i, 0))],
        out_specs=[
            pl.BlockSpec((BLK, 1), lambda i: (i, 0)),
            pl.BlockSpec((1, C), lambda i: (0, 0)),
        ],
        out_shape=[
            jax.ShapeDtypeStruct((N, 1), jnp.int32),
            jax.ShapeDtypeStruct((1, C), jnp.float32),
        ],
        compiler_params=pltpu.CompilerParams(
            dimension_semantics=("arbitrary",)),
    )(logits)

    labels = labels2d.reshape(N)
    zeros = jnp.zeros((C, D), jnp.float32)
    sums2 = _sc_segment_sum(img_feats, labels, zeros)

    out = pl.pallas_call(
        _loss_body,
        out_shape=jax.ShapeDtypeStruct((1, 1), jnp.float32),
    )(sums2, counts)
    return out[0, 0]


# speedup vs baseline: 1.2012x; 1.2012x over previous
import functools

import jax
import jax.numpy as jnp
from jax import lax
from jax.experimental import pallas as pl
from jax.experimental.pallas import tpu as pltpu
from jax.experimental.pallas import tpu_sc as plsc

N = 100000
C = 100
D = 128
BLK = 5000

NC = 2
NS = 16
NW = NC * NS
CHUNK = 128
CA = 128
FULL_CHUNKS = N // CHUNK
TAIL = N - FULL_CHUNKS * CHUNK
BASE_T = FULL_CHUNKS // NW
EXTRA_W = FULL_CHUNKS % NW
NBUF = 4


def _argmax_body(logits_ref, labels_ref, counts_ref):
    i = pl.program_id(0)
    x = logits_ref[...]
    lbl = jnp.argmax(x, axis=1).astype(jnp.int32)
    labels_ref[...] = lbl[:, None]
    iota = lax.broadcasted_iota(jnp.int32, (1, C), 1)
    onehot = (lbl[:, None] == iota).astype(jnp.float32)
    pcounts = jnp.sum(onehot, axis=0)[None, :]

    @pl.when(i == 0)
    def _():
        counts_ref[...] = pcounts

    @pl.when(i > 0)
    def _():
        counts_ref[...] += pcounts


def _sc_seg_body(feats_hbm, labels_hbm, zf_hbm, zi_hbm, out_hbm,
                 rows_v, idx_v, trows_v, tidx_v, acc_sh, gsem, isem, ssem):
    cid = lax.axis_index("c")
    sid = lax.axis_index("s")
    w = sid * NC + cid

    @pl.when(sid == 0)
    def _():
        pltpu.sync_copy(zf_hbm, acc_sh)

    plsc.subcore_barrier()

    def gather(t):
        b = t % NBUF
        base = (w + NW * t) * CHUNK
        gf = pltpu.async_copy(
            feats_hbm.at[pl.ds(base, CHUNK), :], rows_v.at[b], gsem.at[b])
        gi = pltpu.async_copy(
            labels_hbm.at[pl.ds(base, CHUNK)], idx_v.at[b], isem.at[b])
        return gf, gi

    def scatter_add(t):
        b = t % NBUF
        return pltpu.async_copy(
            rows_v.at[b], acc_sh.at[idx_v.at[b]], ssem.at[b], add=True)

    g = [None] * BASE_T
    s = [None] * BASE_T
    g[0] = gather(0)
    g[1] = gather(1)
    for t in range(BASE_T):
        nxt = t + 2
        if nxt < BASE_T:
            if nxt - NBUF >= 0:
                s[nxt - NBUF].wait()
            g[nxt] = gather(nxt)
        g[t][0].wait()
        g[t][1].wait()
        s[t] = scatter_add(t)
    for t in range(BASE_T - NBUF, BASE_T):
        s[t].wait()

    @pl.when(w < EXTRA_W)
    def _():
        base = (w + NW * BASE_T) * CHUNK
        pltpu.sync_copy(feats_hbm.at[pl.ds(base, CHUNK), :], rows_v.at[0])
        pltpu.sync_copy(labels_hbm.at[pl.ds(base, CHUNK)], idx_v.at[0])
        pltpu.sync_copy(rows_v.at[0], acc_sh.at[idx_v.at[0]], add=True)

    @pl.when(w == NW - 1)
    def _():
        base = FULL_CHUNKS * CHUNK
        pltpu.sync_copy(zf_hbm, trows_v)
        pltpu.sync_copy(zi_hbm, tidx_v)
        pltpu.sync_copy(feats_hbm.at[pl.ds(base, TAIL), :],
                        trows_v.at[pl.ds(0, TAIL), :])
        pltpu.sync_copy(labels_hbm.at[pl.ds(base, TAIL)],
                        tidx_v.at[pl.ds(0, TAIL)])
        pltpu.sync_copy(trows_v, acc_sh.at[tidx_v], add=True)

    plsc.subcore_barrier()

    @pl.when(sid == 0)
    def _():
        pltpu.sync_copy(acc_sh, out_hbm.at[cid])


_sc_segment_sum = functools.partial(
    pl.kernel,
    _sc_seg_body,
    out_type=jax.ShapeDtypeStruct((NC, CA, D), jnp.float32),
    mesh=plsc.VectorSubcoreMesh(core_axis_name="c", subcore_axis_name="s"),
    scratch_types=[
        pltpu.VMEM((NBUF, CHUNK, D), jnp.float32),
        pltpu.VMEM((NBUF, CHUNK), jnp.int32),
        pltpu.VMEM((CHUNK, D), jnp.float32),
        pltpu.VMEM((CHUNK,), jnp.int32),
        pltpu.VMEM_SHARED((CA, D), jnp.float32),
        pltpu.SemaphoreType.DMA((NBUF,)),
        pltpu.SemaphoreType.DMA((NBUF,)),
        pltpu.SemaphoreType.DMA((NBUF,)),
    ],
)()


def _loss_body(sums_ref, counts_ref, out_ref):
    cnt = counts_ref[0, :]
    sums = (sums_ref[0] + sums_ref[1])[0:C, :]
    recip = 1.0 / jnp.maximum(cnt, 1.0)
    means = sums * recip[:, None]
    sq = jnp.sum(means * means, axis=1, keepdims=True)
    norm = jnp.maximum(jnp.sqrt(sq), 1e-12)
    normed = means / norm
    cos = lax.dot_general(
        normed, normed, (((1,), (1,)), ((), ())),
        preferred_element_type=jnp.float32)
    present = (cnt > 0.0).astype(jnp.float32)
    pm = present[:, None] * present[None, :]
    ri = lax.broadcasted_iota(jnp.int32, (C, C), 0)
    ci = lax.broadcasted_iota(jnp.int32, (C, C), 1)
    offdiag = (ri != ci).astype(jnp.float32)
    loss = (1.0 - cos) * pm * offdiag
    out_ref[...] = jnp.sum(loss).reshape(1, 1)


def kernel(logits, img_feats):
    labels2d, counts = pl.pallas_call(
        _argmax_body,
        grid=(N // BLK,),
        in_specs=[pl.BlockSpec((BLK, C), lambda i: (i, 0))],
        out_specs=[
            pl.BlockSpec((BLK, 1), lambda i: (i, 0)),
            pl.BlockSpec((1, C), lambda i: (0, 0)),
        ],
        out_shape=[
            jax.ShapeDtypeStruct((N, 1), jnp.int32),
            jax.ShapeDtypeStruct((1, C), jnp.float32),
        ],
        compiler_params=pltpu.CompilerParams(
            dimension_semantics=("arbitrary",)),
    )(logits)

    labels = labels2d.reshape(N)
    zf = jnp.zeros((CA, D), jnp.float32)
    zi = jnp.zeros((CHUNK,), jnp.int32)
    sums2 = _sc_segment_sum(img_feats, labels, zf, zi)

    out = pl.pallas_call(
        _loss_body,
        out_shape=jax.ShapeDtypeStruct((1, 1), jnp.float32),
    )(sums2, counts)
    return out[0, 0]


# speedup vs baseline: 2.2792x; 1.8975x over previous
import functools

import jax
import jax.numpy as jnp
from jax import lax
from jax.experimental import pallas as pl
from jax.experimental.pallas import tpu as pltpu
from jax.experimental.pallas import tpu_sc as plsc

N = 100000
C = 100
D = 128
BLK = 4096
GRID_A = (N + BLK - 1) // BLK

NC = 2
NS = 16
NW = NC * NS
CHUNK = 128
CA = 128
FULL_CHUNKS = N // CHUNK
TAIL = N - FULL_CHUNKS * CHUNK
BASE_T = FULL_CHUNKS // NW
EXTRA_W = FULL_CHUNKS % NW
NBUF = 4


def _argmax_body(logits_ref, labels_ref, counts_ref):
    i = pl.program_id(0)
    x = logits_ref[...]
    lbl = jnp.argmax(x, axis=0).astype(jnp.int32)[None, :]
    col = i * BLK + lax.broadcasted_iota(jnp.int32, (1, BLK), 1)
    valid = col < N
    lbl = jnp.where(valid, lbl, 0)
    labels_ref[...] = lbl[None]
    iota = lax.broadcasted_iota(jnp.int32, (C, 1), 0)
    onehot = jnp.where(valid, (lbl == iota).astype(jnp.float32), 0.0)
    pcounts = jnp.sum(onehot, axis=1, keepdims=True)

    @pl.when(i == 0)
    def _():
        counts_ref[...] = pcounts

    @pl.when(i > 0)
    def _():
        counts_ref[...] += pcounts


def _sc_seg_body(feats_hbm, labels_hbm, zf_hbm, out_hbm,
                 rows_v, idx_v, trows_v, tidx_v, acc_sh, gsem, isem, ssem):
    cid = lax.axis_index("c")
    sid = lax.axis_index("s")
    w = sid * NC + cid

    @pl.when(sid == 0)
    def _():
        pltpu.sync_copy(zf_hbm, acc_sh)

    plsc.subcore_barrier()

    def gather(t):
        b = t % NBUF
        base = (w + NW * t) * CHUNK
        gf = pltpu.async_copy(
            feats_hbm.at[pl.ds(base, CHUNK), :], rows_v.at[b], gsem.at[b])
        gi = pltpu.async_copy(
            labels_hbm.at[pl.ds(base, CHUNK)], idx_v.at[b], isem.at[b])
        return gf, gi

    def scatter_add(t):
        b = t % NBUF
        return pltpu.async_copy(
            rows_v.at[b], acc_sh.at[idx_v.at[b]], ssem.at[b], add=True)

    g = [None] * BASE_T
    s = [None] * BASE_T
    g[0] = gather(0)
    g[1] = gather(1)
    for t in range(BASE_T):
        nxt = t + 2
        if nxt < BASE_T:
            if nxt - NBUF >= 0:
                s[nxt - NBUF].wait()
            g[nxt] = gather(nxt)
        g[t][0].wait()
        g[t][1].wait()
        s[t] = scatter_add(t)
    for t in range(BASE_T - NBUF, BASE_T):
        s[t].wait()

    @pl.when(w < EXTRA_W)
    def _():
        base = (w + NW * BASE_T) * CHUNK
        pltpu.sync_copy(feats_hbm.at[pl.ds(base, CHUNK), :], rows_v.at[0])
        pltpu.sync_copy(labels_hbm.at[pl.ds(base, CHUNK)], idx_v.at[0])
        pltpu.sync_copy(rows_v.at[0], acc_sh.at[idx_v.at[0]], add=True)

    @pl.when(w == NW - 1)
    def _():
        base = FULL_CHUNKS * CHUNK
        pltpu.sync_copy(zf_hbm, trows_v)
        pltpu.sync_copy(feats_hbm.at[pl.ds(base, TAIL), :],
                        trows_v.at[pl.ds(0, TAIL), :])
        pltpu.sync_copy(labels_hbm.at[pl.ds(base, CHUNK)], tidx_v)
        pltpu.sync_copy(trows_v, acc_sh.at[tidx_v], add=True)

    plsc.subcore_barrier()

    @pl.when(sid == 0)
    def _():
        pltpu.sync_copy(acc_sh, out_hbm.at[cid])


_sc_segment_sum = functools.partial(
    pl.kernel,
    _sc_seg_body,
    out_type=jax.ShapeDtypeStruct((NC, CA, D), jnp.float32),
    mesh=plsc.VectorSubcoreMesh(core_axis_name="c", subcore_axis_name="s"),
    scratch_types=[
        pltpu.VMEM((NBUF, CHUNK, D), jnp.float32),
        pltpu.VMEM((NBUF, CHUNK), jnp.int32),
        pltpu.VMEM((CHUNK, D), jnp.float32),
        pltpu.VMEM((CHUNK,), jnp.int32),
        pltpu.VMEM_SHARED((CA, D), jnp.float32),
        pltpu.SemaphoreType.DMA((NBUF,)),
        pltpu.SemaphoreType.DMA((NBUF,)),
        pltpu.SemaphoreType.DMA((NBUF,)),
    ],
)()


def _loss_body(sums_ref, counts_ref, out_ref):
    cnt = counts_ref[...]
    sums = (sums_ref[0] + sums_ref[1])[0:C, :]
    recip = 1.0 / jnp.maximum(cnt, 1.0)
    means = sums * recip
    sq = jnp.sum(means * means, axis=1, keepdims=True)
    norm = jnp.maximum(jnp.sqrt(sq), 1e-12)
    normed = means / norm
    cos = lax.dot_general(
        normed, normed, (((1,), (1,)), ((), ())),
        preferred_element_type=jnp.float32)
    present = (cnt > 0.0).astype(jnp.float32)
    pm = lax.dot_general(
        present, present, (((1,), (1,)), ((), ())),
        preferred_element_type=jnp.float32)
    ri = lax.broadcasted_iota(jnp.int32, (C, C), 0)
    ci = lax.broadcasted_iota(jnp.int32, (C, C), 1)
    offdiag = (ri != ci).astype(jnp.float32)
    loss = (1.0 - cos) * pm * offdiag
    out_ref[...] = jnp.sum(loss).reshape(1, 1)


def kernel(logits, img_feats):
    labels2d, counts = pl.pallas_call(
        _argmax_body,
        grid=(GRID_A,),
        in_specs=[pl.BlockSpec((C, BLK), lambda i: (0, i))],
        out_specs=[
            pl.BlockSpec((1, 1, BLK), lambda i: (i, 0, 0)),
            pl.BlockSpec((C, 1), lambda i: (0, 0)),
        ],
        out_shape=[
            jax.ShapeDtypeStruct((GRID_A, 1, BLK), jnp.int32),
            jax.ShapeDtypeStruct((C, 1), jnp.float32),
        ],
        compiler_params=pltpu.CompilerParams(
            dimension_semantics=("arbitrary",)),
    )(logits.T)

    labels = labels2d.reshape(GRID_A * BLK)
    zf = jnp.zeros((CA, D), jnp.float32)
    sums2 = _sc_segment_sum(img_feats, labels, zf)

    out = pl.pallas_call(
        _loss_body,
        out_shape=jax.ShapeDtypeStruct((1, 1), jnp.float32),
    )(sums2, counts)
    return out[0, 0]


# speedup vs baseline: 2.3837x; 1.0458x over previous
import functools

import jax
import jax.numpy as jnp
from jax import lax
from jax.experimental import pallas as pl
from jax.experimental.pallas import tpu as pltpu
from jax.experimental.pallas import tpu_sc as plsc

N = 100000
C = 100
D = 128
BLK = 4096

NC = 2
NS = 16
NW = NC * NS
CHUNK = 128
CA = 128
NBUF = 4

P0_BLOCKS = 12
P1_BLOCKS = 13
P0_ROWS = P0_BLOCKS * BLK


def _argmax_body(part_base, logits_ref, labels_ref, counts_ref):
    i = pl.program_id(0)
    x = logits_ref[...]
    lbl = jnp.argmax(x, axis=0).astype(jnp.int32)[None, :]
    col = part_base + i * BLK + lax.broadcasted_iota(jnp.int32, (1, BLK), 1)
    valid = col < N
    lbl = jnp.where(valid, lbl, 0)
    labels_ref[...] = lbl[None]
    iota = lax.broadcasted_iota(jnp.int32, (C, 1), 0)
    onehot = jnp.where(valid, (lbl == iota).astype(jnp.float32), 0.0)
    pcounts = jnp.sum(onehot, axis=1, keepdims=True)

    @pl.when(i == 0)
    def _():
        counts_ref[...] = pcounts

    @pl.when(i > 0)
    def _():
        counts_ref[...] += pcounts


def _argmax_call(logits_t, part_blk0, nblocks):
    return pl.pallas_call(
        functools.partial(_argmax_body, part_blk0 * BLK),
        grid=(nblocks,),
        in_specs=[pl.BlockSpec((C, BLK), lambda i: (0, part_blk0 + i))],
        out_specs=[
            pl.BlockSpec((1, 1, BLK), lambda i: (i, 0, 0)),
            pl.BlockSpec((C, 1), lambda i: (0, 0)),
        ],
        out_shape=[
            jax.ShapeDtypeStruct((nblocks, 1, BLK), jnp.int32),
            jax.ShapeDtypeStruct((C, 1), jnp.float32),
        ],
        compiler_params=pltpu.CompilerParams(
            dimension_semantics=("arbitrary",)),
    )(logits_t)


def _sc_seg_body(base, base_t, extra_w, tail_rows, tail_chunk,
                 feats_hbm, labels_hbm, zf_hbm, out_hbm,
                 rows_v, idx_v, trows_v, tidx_v, acc_sh, gsem, isem, ssem):
    cid = lax.axis_index("c")
    sid = lax.axis_index("s")
    w = sid * NC + cid

    @pl.when(sid == 0)
    def _():
        pltpu.sync_copy(zf_hbm, acc_sh)

    plsc.subcore_barrier()

    def gather(t):
        b = t % NBUF
        ch = w + NW * t
        gf = pltpu.async_copy(
            feats_hbm.at[pl.ds(base + ch * CHUNK, CHUNK), :],
            rows_v.at[b], gsem.at[b])
        gi = pltpu.async_copy(
            labels_hbm.at[pl.ds(ch * CHUNK, CHUNK)], idx_v.at[b],
            isem.at[b])
        return gf, gi

    def scatter_add(t):
        b = t % NBUF
        return pltpu.async_copy(
            rows_v.at[b], acc_sh.at[idx_v.at[b]], ssem.at[b], add=True)

    g = [None] * base_t
    s = [None] * base_t
    g[0] = gather(0)
    g[1] = gather(1)
    for t in range(base_t):
        nxt = t + 2
        if nxt < base_t:
            if nxt - NBUF >= 0:
                s[nxt - NBUF].wait()
            g[nxt] = gather(nxt)
        g[t][0].wait()
        g[t][1].wait()
        s[t] = scatter_add(t)
    for t in range(max(0, base_t - NBUF), base_t):
        s[t].wait()

    if extra_w:
        @pl.when(w < extra_w)
        def _():
            ch = w + NW * base_t
            pltpu.sync_copy(feats_hbm.at[pl.ds(base + ch * CHUNK, CHUNK), :],
                            rows_v.at[0])
            pltpu.sync_copy(labels_hbm.at[pl.ds(ch * CHUNK, CHUNK)],
                            idx_v.at[0])
            pltpu.sync_copy(rows_v.at[0], acc_sh.at[idx_v.at[0]], add=True)

    if tail_rows:
        @pl.when(w == NW - 1)
        def _():
            pltpu.sync_copy(zf_hbm, trows_v)
            pltpu.sync_copy(
                feats_hbm.at[pl.ds(base + tail_chunk * CHUNK, tail_rows), :],
                trows_v.at[pl.ds(0, tail_rows), :])
            pltpu.sync_copy(labels_hbm.at[pl.ds(tail_chunk * CHUNK, CHUNK)],
                            tidx_v)
            pltpu.sync_copy(trows_v, acc_sh.at[tidx_v], add=True)

    plsc.subcore_barrier()

    @pl.when(sid == 0)
    def _():
        pltpu.sync_copy(acc_sh, out_hbm.at[cid])


def _make_sc(base, base_t, extra_w, tail_rows, tail_chunk):
    return functools.partial(
        pl.kernel,
        functools.partial(_sc_seg_body, base, base_t, extra_w,
                          tail_rows, tail_chunk),
        out_type=jax.ShapeDtypeStruct((NC, CA, D), jnp.float32),
        mesh=plsc.VectorSubcoreMesh(core_axis_name="c", subcore_axis_name="s"),
        scratch_types=[
            pltpu.VMEM((NBUF, CHUNK, D), jnp.float32),
            pltpu.VMEM((NBUF, CHUNK), jnp.int32),
            pltpu.VMEM((CHUNK, D), jnp.float32),
            pltpu.VMEM((CHUNK,), jnp.int32),
            pltpu.VMEM_SHARED((CA, D), jnp.float32),
            pltpu.SemaphoreType.DMA((NBUF,)),
            pltpu.SemaphoreType.DMA((NBUF,)),
            pltpu.SemaphoreType.DMA((NBUF,)),
        ],
    )()


_sc_part0 = _make_sc(0, 12, 0, 0, 0)
_sc_part1 = _make_sc(P0_ROWS, 12, 13, 32, 397)


def _loss_body(sums_a, sums_b, counts_a, counts_b, out_ref):
    cnt = counts_a[...] + counts_b[...]
    sums = (sums_a[0] + sums_a[1] + sums_b[0] + sums_b[1])[0:C, :]
    recip = 1.0 / jnp.maximum(cnt, 1.0)
    means = sums * recip
    sq = jnp.sum(means * means, axis=1, keepdims=True)
    norm = jnp.maximum(jnp.sqrt(sq), 1e-12)
    normed = means / norm
    cos = lax.dot_general(
        normed, normed, (((1,), (1,)), ((), ())),
        preferred_element_type=jnp.float32)
    present = (cnt > 0.0).astype(jnp.float32)
    pm = lax.dot_general(
        present, present, (((1,), (1,)), ((), ())),
        preferred_element_type=jnp.float32)
    ri = lax.broadcasted_iota(jnp.int32, (C, C), 0)
    ci = lax.broadcasted_iota(jnp.int32, (C, C), 1)
    offdiag = (ri != ci).astype(jnp.float32)
    loss = (1.0 - cos) * pm * offdiag
    out_ref[...] = jnp.sum(loss).reshape(1, 1)


def kernel(logits, img_feats):
    lt = logits.T
    labels_a, counts_a = _argmax_call(lt, 0, P0_BLOCKS)
    labels_b, counts_b = _argmax_call(lt, P0_BLOCKS, P1_BLOCKS)
    zf = jnp.zeros((CA, D), jnp.float32)
    sums_a = _sc_part0(img_feats, labels_a.reshape(P0_BLOCKS * BLK), zf)
    sums_b = _sc_part1(img_feats, labels_b.reshape(P1_BLOCKS * BLK), zf)

    out = pl.pallas_call(
        _loss_body,
        out_shape=jax.ShapeDtypeStruct((1, 1), jnp.float32),
    )(sums_a, sums_b, counts_a, counts_b)
    return out[0, 0]
